# baseline (speedup 1.0000x reference)
import functools

import jax
import jax.numpy as jnp
from jax import lax
from jax.experimental import pallas as pl
from jax.experimental.pallas import tpu as pltpu
from jax.experimental.pallas import tpu_sc as plsc

N = 50000
E = 800000
D = 64
HALF = 32
B = 16
TD = 16
L = 2
NLAYER_TYPES = 5
SRC_TYPE = (0, 1, 0, 2, 0)
C_EDGE = 2000
N_CHUNK = (E // 16) // C_EDGE
QUART = 16
CS = 1000
NCH0 = (E // 16) // CS
STRIPE_A = 3128
SUBCHUNKS_A = ((0, 1000), (1000, 1000), (2000, 1000), (3000, 128))
SUBCHUNKS_B = ((0, 1000), (1000, 1000), (2000, 1000), (3000, 80))

_MESH = plsc.VectorSubcoreMesh(core_axis_name="c", subcore_axis_name="s")


def _sc_counts_body(d0, d1, d2, d3, d4, o0, o1, o2, o3, o4, acc, idst, ones, zb, bounce):
    c = lax.axis_index("c")
    s = lax.axis_index("s")
    dsts = [d0, d1, d2, d3, d4]
    outs = [o0, o1, o2, o3, o4]

    def fill(i, _):
        ones[pl.ds(i * 16, 16)] = jnp.ones((16,), jnp.float32)
        zb[pl.ds(i * 16, 16)] = jnp.zeros((16,), jnp.float32)
        return 0

    lax.fori_loop(0, 3200 // 16, fill, 0)

    for e in range(NLAYER_TYPES):
        @pl.when(c == e % 2)
        def _():
            @pl.when(s < 15)
            def _():
                pltpu.sync_copy(zb, acc.at[pl.ds(s * 3200, 3200)])

            @pl.when(s == 15)
            def _():
                pltpu.sync_copy(zb.at[pl.ds(0, 2000)], acc.at[pl.ds(48000, 2000)])

            plsc.subcore_barrier()

            def chunk(j, _):
                off = s * (E // 16) + j * C_EDGE
                pltpu.sync_copy(dsts[e].at[pl.ds(off, C_EDGE)], idst)
                pltpu.sync_copy(ones.at[pl.ds(0, C_EDGE)], acc.at[idst], add=True)
                return 0

            lax.fori_loop(0, N_CHUNK, chunk, 0)
            plsc.subcore_barrier()

            @pl.when(s < 15)
            def _():
                pltpu.sync_copy(acc.at[pl.ds(s * 3200, 3200)], bounce)
                pltpu.sync_copy(bounce, outs[e].at[pl.ds(s * 3200, 3200)])

            @pl.when(s == 15)
            def _():
                pltpu.sync_copy(acc.at[pl.ds(48000, 2000)], bounce.at[pl.ds(0, 2000)])
                pltpu.sync_copy(bounce.at[pl.ds(0, 2000)], outs[e].at[pl.ds(48000, 2000)])

            plsc.subcore_barrier()


def _sc_counts(d0, d1, d2, d3, d4):
    return pl.kernel(
        _sc_counts_body,
        compiler_params=pltpu.CompilerParams(use_tc_tiling_on_sc=False),
        out_type=tuple(jax.ShapeDtypeStruct((N,), jnp.float32) for _ in range(NLAYER_TYPES)),
        mesh=_MESH,
        scratch_types=[
            pltpu.VMEM_SHARED((N,), jnp.float32),
            pltpu.VMEM((C_EDGE,), jnp.int32),
            pltpu.VMEM((3200,), jnp.float32),
            pltpu.VMEM((3200,), jnp.float32),
            pltpu.VMEM((3200,), jnp.float32),
        ],
    )(d0, d1, d2, d3, d4)


TSEL = (0, 0, 0, 1, 0)
GBASE = (0, 4, 0, 0, 0)
SMAP = ((1, 4), (0, 0), (0, 4), (1, 0), (2, 0))


def _sc_scatter_body(tA, tB, s0, s1, s2, s3, s4,
                     oA, oB, oC, acc,
                     ibA, ibB, ibC,
                     rowsA, rowsB, rowsC,
                     sgA, sgB, sgC, ssA, ssB, ssC):
    c = lax.axis_index("c")
    s = lax.axis_index("s")
    tables = [tA, tB]
    combos = [s0, s1, s2, s3, s4]
    outs = [oA, oB, oC]
    ib = [ibA, ibB, ibC]
    rows = [rowsA, rowsB, rowsC]
    sg = [sgA, sgB, sgC]
    ss = [ssA, ssB, ssC]
    NCH = (E // 16) // CS
    NTRIP = (NCH - 2) // 3

    def stripe_io(body):
        @pl.when(s < 15)
        def _():
            for off, sz in SUBCHUNKS_A:
                body(s * STRIPE_A + off, sz)

        @pl.when(s == 15)
        def _():
            for off, sz in SUBCHUNKS_B:
                body(15 * STRIPE_A + off, sz)

    for e in range(NLAYER_TYPES):
        table = tables[TSEL[e]]
        oi, goff = SMAP[e]
        for q in range(2):
            qc = 2 * c + q
            sbase = qc * (16 * NCH0) + s * NCH0
            g = goff + qc

            def fillz(i, _):
                rowsA[i, pl.ds(0, 16)] = jnp.zeros((16,), jnp.float32)
                return 0

            lax.fori_loop(0, CS, fillz, 0)

            def zero(off, sz):
                pltpu.sync_copy(rowsA.at[pl.ds(0, sz)], acc.at[pl.ds(off, sz)])

            stripe_io(zero)
            plsc.subcore_barrier()

            def load_idx(j, b):
                pltpu.sync_copy(combos[e].at[sbase + j], ib[b])

            def gather_start(b):
                pltpu.async_copy(table.at[ib[b].at[1]], rows[b], sg[b])

            def gather_wait(b):
                pltpu.make_async_copy(table.at[ib[b].at[1]], rows[b], sg[b]).wait()

            def scatter_start(b):
                pltpu.async_copy(rows[b], acc.at[ib[b].at[0]], ss[b], add=True)

            def scatter_wait(b):
                pltpu.make_async_copy(rows[b], acc.at[ib[b].at[0]], ss[b]).wait()

            load_idx(0, 0)
            gather_start(0)

            def slot(p, b, guard):
                nb = (b + 1) % 3
                if guard:
                    @pl.when(p > 0)
                    def _():
                        scatter_wait(nb)
                else:
                    scatter_wait(nb)
                load_idx(3 * p + b + 1, nb)
                gather_start(nb)
                gather_wait(b)
                scatter_start(b)

            def triple(p, _):
                slot(p, 0, True)
                slot(p, 1, True)
                slot(p, 2, False)
                return 0

            lax.fori_loop(0, NTRIP, triple, 0)
            for j in range(3 * NTRIP, NCH - 1):
                b = j % 3
                nb = (j + 1) % 3
                scatter_wait(nb)
                load_idx(j + 1, nb)
                gather_start(nb)
                gather_wait(b)
                scatter_start(b)
            bl_ = (NCH - 1) % 3
            gather_wait(bl_)
            scatter_start(bl_)
            scatter_wait((NCH - 3) % 3)
            scatter_wait((NCH - 2) % 3)
            scatter_wait(bl_)
            plsc.subcore_barrier()

            def outcopy(off, sz):
                pltpu.sync_copy(acc.at[pl.ds(off, sz)], rowsA.at[pl.ds(0, sz)])
                pltpu.sync_copy(rowsA.at[pl.ds(0, sz)],
                                outs[oi].at[pl.ds(off, sz), pl.ds(16 * g, 16)])

            stripe_io(outcopy)
            plsc.subcore_barrier()


def _sc_scatter(tA, tB, combos):
    return pl.kernel(
        _sc_scatter_body,
        compiler_params=pltpu.CompilerParams(use_tc_tiling_on_sc=False),
        out_type=tuple(jax.ShapeDtypeStruct((N, 128), jnp.float32) for _ in range(3)),
        mesh=_MESH,
        scratch_types=[
            pltpu.VMEM_SHARED((N, QUART), jnp.float32),
            pltpu.VMEM((2, CS), jnp.int32),
            pltpu.VMEM((2, CS), jnp.int32),
            pltpu.VMEM((2, CS), jnp.int32),
            pltpu.VMEM((CS, QUART), jnp.float32),
            pltpu.VMEM((CS, QUART), jnp.float32),
            pltpu.VMEM((CS, QUART), jnp.float32),
            pltpu.SemaphoreType.DMA,
            pltpu.SemaphoreType.DMA,
            pltpu.SemaphoreType.DMA,
            pltpu.SemaphoreType.DMA,
            pltpu.SemaphoreType.DMA,
            pltpu.SemaphoreType.DMA,
        ],
    )(tA, tB, *combos)


R_POST = 2000


def _tc_post_kernel(sa_ref, sb_ref, sc_ref, xa_ref, xb_ref, cnt_ref, w_ref,
                    b_ref, oa_ref, ob_ref):
    inv = 1.0 / jnp.maximum(cnt_ref[...], 1.0)
    lane = lax.broadcasted_iota(jnp.int32, (R_POST, 128), 1)
    low = lane < 64

    def scaled(ref, el, eh):
        pat = jnp.where(low, inv[:, el:el + 1], inv[:, eh:eh + 1])
        return ref[...] * pat

    sa = scaled(sa_ref, 1, 2)
    sb = scaled(sb_ref, 3, 0)
    sc = jnp.where(low, sc_ref[...] * inv[:, 4:5], 0.0)

    def mm(x, k):
        return jnp.dot(x, w_ref[k], preferred_element_type=jnp.float32)

    z_ch = mm(sa, 0) + mm(sb, 1)
    z_gw = mm(sb, 2)
    z_hru = mm(sc, 3)
    r_gw = mm(xb_ref[...], 4)
    r_ch = mm(xa_ref[...], 5)
    r_hru = mm(xa_ref[...], 6)
    hru = jnp.maximum(z_hru + r_hru + b_ref[0:1, :], 0.0)
    ch = jnp.maximum((z_ch + r_ch + b_ref[1:2, :]) / 3.0, 0.0)
    gw = jnp.maximum(z_gw + r_gw + b_ref[2:3, :], 0.0)
    oa_ref[...] = jnp.concatenate([hru, ch], axis=1)
    ob_ref[...] = jnp.concatenate([gw, hru], axis=1)


def _tc_post(SA, SB, SC2, XA, XB, cntT, w, bsum):
    grid = (N // R_POST,)
    blk = pl.BlockSpec((R_POST, 128), lambda i: (i, 0))
    return pl.pallas_call(
        _tc_post_kernel,
        grid=grid,
        in_specs=[
            blk, blk, blk, blk, blk,
            pl.BlockSpec((R_POST, NLAYER_TYPES), lambda i: (i, 0)),
            pl.BlockSpec((7, 128, D), lambda i: (0, 0, 0)),
            pl.BlockSpec((3, D), lambda i: (0, 0)),
        ],
        out_specs=[blk, blk],
        out_shape=[jax.ShapeDtypeStruct((N, 128), jnp.float32),
                   jax.ShapeDtypeStruct((N, 128), jnp.float32)],
    )(SA, SB, SC2, XA, XB, cntT, w, bsum)


R_POOL = 2000


def _tc_pool_kernel(xa_ref, b_ref, w1p_ref, w1t_ref, td_ref, b1_ref, w2_ref,
                    o_ref, accp, accc):
    i = pl.program_id(0)

    @pl.when(i == 0)
    def _():
        accp[...] = jnp.zeros_like(accp)
        accc[...] = jnp.zeros_like(accc)

    ids = b_ref[0]
    iota = lax.broadcasted_iota(jnp.int32, (B, R_POOL), 0)
    m = (iota == ids).astype(jnp.float32)
    accp[...] += jnp.dot(m, xa_ref[...], preferred_element_type=jnp.float32)
    accc[...] = accc[...] + jnp.sum(m, axis=1, keepdims=True)

    @pl.when(i == (N // R_POOL) - 1)
    def _():
        pooled = accp[...] / jnp.maximum(accc[...], 1.0)
        h = jnp.maximum(jnp.dot(pooled, w1p_ref[...], preferred_element_type=jnp.float32)
                        + jnp.dot(td_ref[...], w1t_ref[...], preferred_element_type=jnp.float32)
                        + b1_ref[...], 0.0)
        o_ref[...] = h * w2_ref[...]


def _tc_pool_head(XA, batch3, w1p, w1t, train_data, b1b, w2):
    grid = (N // R_POOL,)
    return pl.pallas_call(
        _tc_pool_kernel,
        grid=grid,
        in_specs=[
            pl.BlockSpec((R_POOL, 128), lambda i: (i, 0)),
            pl.BlockSpec((1, 1, R_POOL), lambda i: (i, 0, 0)),
            pl.BlockSpec((128, 128), lambda i: (0, 0)),
            pl.BlockSpec((TD, 128), lambda i: (0, 0)),
            pl.BlockSpec((B, TD), lambda i: (0, 0)),
            pl.BlockSpec((B, 128), lambda i: (0, 0)),
            pl.BlockSpec((1, 128), lambda i: (0, 0)),
        ],
        out_specs=pl.BlockSpec((B, 128), lambda i: (0, 0)),
        out_shape=jax.ShapeDtypeStruct((B, 128), jnp.float32),
        scratch_shapes=[
            pltpu.VMEM((B, 128), jnp.float32),
            pltpu.VMEM((B, 128), jnp.float32),
        ],
    )(XA, batch3, w1p, w1t, train_data, b1b, w2)


def kernel(x_hru, x_channel, x_gw_cell, ei_sw_gw, ei_hydro, ei_sw, ei_gw_sw,
           ei_self, batch, train_data, Wl, bl, Wr, fc1_w, fc1_b, fc2_w, fc2_b):
    eis = [ei_sw_gw, ei_hydro, ei_sw, ei_gw_sw, ei_self]
    srcs = [ei[0] for ei in eis]
    dsts = [ei[1] for ei in eis]

    XA = jnp.concatenate([x_hru, x_channel], axis=1)
    XB = jnp.concatenate([x_gw_cell, x_hru], axis=1)

    combos = []
    for e in range(NLAYER_TYPES):
        sa = jnp.stack([8 * srcs[e] + (GBASE[e] + qc) for qc in range(4)])
        sa = sa.reshape(4, 16 * NCH0, CS)
        db = jnp.broadcast_to(dsts[e].reshape(1, 16 * NCH0, CS), (4, 16 * NCH0, CS))
        combos.append(jnp.stack([db, sa], axis=2).reshape(4 * 16 * NCH0, 2, CS))

    cnts = _sc_counts(*dsts)
    cntT = jnp.stack(cnts, axis=1)

    z64 = jnp.zeros((64, 64), jnp.float32)

    def wstack(l):
        wr_ch = (Wr[l, 1] + Wr[l, 2] + Wr[l, 3]).T
        return jnp.stack([
            jnp.concatenate([Wl[l, 1].T, Wl[l, 2].T], axis=0),
            jnp.concatenate([Wl[l, 3].T, z64], axis=0),
            jnp.concatenate([z64, Wl[l, 0].T], axis=0),
            jnp.concatenate([Wl[l, 4].T, z64], axis=0),
            jnp.concatenate([Wr[l, 0].T, z64], axis=0),
            jnp.concatenate([z64, wr_ch], axis=0),
            jnp.concatenate([Wr[l, 4].T, z64], axis=0),
        ])
    w = jnp.stack([wstack(l) for l in range(L)])
    bsum = jnp.stack([bl[:, 4], bl[:, 1] + bl[:, 2] + bl[:, 3], bl[:, 0]],
                     axis=1)

    for l in range(L):
        SA, SB, SC2 = _sc_scatter(XA.reshape(8 * N, QUART), XB.reshape(8 * N, QUART),
                                  combos)
        XA, XB = _tc_post(SA, SB, SC2, XA, XB, cntT, w[l], bsum[l])

    batch3 = batch.reshape(N // R_POOL, 1, R_POOL)
    w1p = jnp.concatenate([jnp.zeros((64, 128), jnp.float32), fc1_w.T[:D]], axis=0)
    w1t = fc1_w.T[D:]
    hw = _tc_pool_head(XA, batch3, w1p, w1t, train_data,
                       jnp.broadcast_to(fc1_b, (B, 128)), fc2_w)
    return jnp.sum(hw, axis=1, keepdims=True) + fc2_b
